# baseline (device time: 96117 ns/iter reference)
import jax
import jax.numpy as jnp
from jax import lax
from jax.experimental import pallas as pl
from jax.experimental.pallas import tpu as pltpu

N_DEV = 4
LAST = N_DEV - 1


def kernel(x, w_mat, scale_x, scale_w):
    m_per, k = x.shape
    _, n_per = w_mat.shape
    half = m_per // 2
    quart = half // 2

    def body(x_ref, w_ref, sx_ref, sw_ref, out_ref,
             buf_a, buf_b, stage, w16, outblk,
             stage_sem, out_sems, send_a, recv_a, send_b, recv_b,
             send_a2, recv_a2, send_b2, recv_b2):
        my = lax.axis_index("i")
        left = lax.rem(my + N_DEV - 1, N_DEV)
        right = lax.rem(my + 1, N_DEV)

        barrier = pltpu.get_barrier_semaphore()
        for nbr in (left, right):
            pl.semaphore_signal(
                barrier, inc=1,
                device_id=(nbr,), device_id_type=pl.DeviceIdType.MESH,
            )
        pl.semaphore_wait(barrier, 2)

        cp_a = pltpu.make_async_copy(
            x_ref.at[pl.ds(0, half), :], stage, stage_sem)
        cp_a.start()
        cp_a.wait()
        buf_a[0] = stage[...].astype(jnp.float8_e4m3fn)

        def make_rdma(h):
            rdma_a = pltpu.make_async_remote_copy(
                src_ref=buf_a.at[h],
                dst_ref=buf_a.at[h + 1],
                send_sem=send_a.at[h],
                recv_sem=recv_a.at[h],
                device_id=(right,),
                device_id_type=pl.DeviceIdType.MESH,
            )
            rdma_b = pltpu.make_async_remote_copy(
                src_ref=buf_b.at[h],
                dst_ref=buf_b.at[h + 1],
                send_sem=send_b.at[h],
                recv_sem=recv_b.at[h],
                device_id=(left,),
                device_id_type=pl.DeviceIdType.MESH,
            )
            return rdma_a, rdma_b

        rdma_a0, rdma_b0 = make_rdma(0)
        rdma_a0.start()

        cp_b = pltpu.make_async_copy(
            x_ref.at[pl.ds(half, half), :], stage, stage_sem)
        cp_b.start()
        cp_b.wait()
        buf_b[0] = stage[...].astype(jnp.float8_e4m3fn)
        rdma_b0.start()

        w16[...] = w_ref[...].astype(jnp.bfloat16)

        scale = sx_ref[0, 0] * sw_ref[0, 0]
        out_copies = []

        def compute_block(buf, s, origin, row0, nrows):
            blk = len(out_copies)
            slot = blk % 2
            if blk >= 2:
                out_copies[blk - 2].wait()
            acc = jnp.dot(buf[s, pl.ds(row0, nrows), :].astype(jnp.bfloat16),
                          w16[...], preferred_element_type=jnp.float32)
            outblk[slot, pl.ds(0, nrows), :] = jnp.maximum(acc * scale, 0.0)
            base = origin * m_per + (0 if buf is buf_a else half)
            cp = pltpu.make_async_copy(
                outblk.at[slot, pl.ds(0, nrows), :],
                out_ref.at[pl.ds(base + row0, nrows), :],
                out_sems.at[slot],
            )
            cp.start()
            out_copies.append(cp)

        def origins(s):
            return (lax.rem(my - s + N_DEV, N_DEV),
                    lax.rem(my + s, N_DEV))

        def compute_slot(s):
            oa, ob = origins(s)
            compute_block(buf_a, s, oa, 0, half)
            compute_block(buf_b, s, ob, 0, half)

        for h in range(N_DEV - 2):
            if h > 0:
                rdma_a, rdma_b = make_rdma(h)
                rdma_a.start()
                rdma_b.start()
            else:
                rdma_a, rdma_b = rdma_a0, rdma_b0
            compute_slot(h)
            rdma_a.wait()
            rdma_b.wait()

        h = N_DEV - 2
        subs = []
        for q, (ss_a, rs_a, ss_b, rs_b) in enumerate(
            ((send_a, recv_a, send_b, recv_b),
             (send_a2, recv_a2, send_b2, recv_b2))
        ):
            row = q * quart
            sub_a = pltpu.make_async_remote_copy(
                src_ref=buf_a.at[h, pl.ds(row, quart), :],
                dst_ref=buf_a.at[h + 1, pl.ds(row, quart), :],
                send_sem=ss_a.at[h],
                recv_sem=rs_a.at[h],
                device_id=(right,),
                device_id_type=pl.DeviceIdType.MESH,
            )
            sub_b = pltpu.make_async_remote_copy(
                src_ref=buf_b.at[h, pl.ds(row, quart), :],
                dst_ref=buf_b.at[h + 1, pl.ds(row, quart), :],
                send_sem=ss_b.at[h],
                recv_sem=rs_b.at[h],
                device_id=(left,),
                device_id_type=pl.DeviceIdType.MESH,
            )
            sub_a.start()
            sub_b.start()
            subs.append((sub_a, sub_b))
        compute_slot(h)
        oa, ob = origins(LAST)
        for q, (sub_a, sub_b) in enumerate(subs):
            sub_a.wait()
            sub_b.wait()
            compute_block(buf_a, LAST, oa, q * quart, quart)
            compute_block(buf_b, LAST, ob, q * quart, quart)

        out_copies[-2].wait()
        out_copies[-1].wait()

    return pl.pallas_call(
        body,
        out_shape=jax.ShapeDtypeStruct((N_DEV * m_per, n_per), jnp.float32),
        in_specs=[
            pl.BlockSpec(memory_space=pl.ANY),
            pl.BlockSpec(memory_space=pltpu.VMEM),
            pl.BlockSpec(memory_space=pltpu.SMEM),
            pl.BlockSpec(memory_space=pltpu.SMEM),
        ],
        out_specs=pl.BlockSpec(memory_space=pl.ANY),
        scratch_shapes=[
            pltpu.VMEM((N_DEV, half, k), jnp.float8_e4m3fn),
            pltpu.VMEM((N_DEV, half, k), jnp.float8_e4m3fn),
            pltpu.VMEM((half, k), jnp.float32),
            pltpu.VMEM((k, n_per), jnp.bfloat16),
            pltpu.VMEM((2, half, n_per), jnp.float32),
            pltpu.SemaphoreType.DMA,
            pltpu.SemaphoreType.DMA((2,)),
            pltpu.SemaphoreType.DMA((N_DEV - 1,)),
            pltpu.SemaphoreType.DMA((N_DEV - 1,)),
            pltpu.SemaphoreType.DMA((N_DEV - 1,)),
            pltpu.SemaphoreType.DMA((N_DEV - 1,)),
            pltpu.SemaphoreType.DMA((N_DEV - 1,)),
            pltpu.SemaphoreType.DMA((N_DEV - 1,)),
            pltpu.SemaphoreType.DMA((N_DEV - 1,)),
            pltpu.SemaphoreType.DMA((N_DEV - 1,)),
        ],
        compiler_params=pltpu.CompilerParams(
            collective_id=0,
            vmem_limit_bytes=100 * 1024 * 1024,
        ),
    )(x, w_mat, scale_x.reshape(1, 1), scale_w.reshape(1, 1))


# device time: 90782 ns/iter; 1.0588x vs baseline; 1.0588x over previous
import jax
import jax.numpy as jnp
from jax import lax
from jax.experimental import pallas as pl
from jax.experimental.pallas import tpu as pltpu

N_DEV = 4
N_HOP = N_DEV - 1
N_SUB = 2


def kernel(x, w_mat, scale_x, scale_w):
    m_per, k = x.shape
    _, n_per = w_mat.shape
    half = m_per // 2
    sub = half // N_SUB

    def body(x_ref, w_ref, sx_ref, sw_ref, out_ref,
             buf_a, buf_b, stage, w16, outblk,
             stage_sem, out_sems, *ring_sems):
        my = lax.axis_index("i")
        left = lax.rem(my + N_DEV - 1, N_DEV)
        right = lax.rem(my + 1, N_DEV)

        dirs = {
            "a": dict(buf=buf_a, peer=right, base=0,
                      origin=lambda s: lax.rem(my - s + N_DEV, N_DEV)),
            "b": dict(buf=buf_b, peer=left, base=half,
                      origin=lambda s: lax.rem(my + s, N_DEV)),
        }
        for q in range(N_SUB):
            for di, d in enumerate(("a", "b")):
                dirs[d].setdefault("send", {})[q] = ring_sems[4 * q + 2 * di]
                dirs[d].setdefault("recv", {})[q] = ring_sems[4 * q + 2 * di + 1]

        barrier = pltpu.get_barrier_semaphore()
        for nbr in (left, right):
            pl.semaphore_signal(
                barrier, inc=1,
                device_id=(nbr,), device_id_type=pl.DeviceIdType.MESH,
            )
        pl.semaphore_wait(barrier, 2)

        def make_rdma(d, h, q):
            dd = dirs[d]
            return pltpu.make_async_remote_copy(
                src_ref=dd["buf"].at[h, pl.ds(q * sub, sub), :],
                dst_ref=dd["buf"].at[h + 1, pl.ds(q * sub, sub), :],
                send_sem=dd["send"][q].at[h],
                recv_sem=dd["recv"][q].at[h],
                device_id=(dd["peer"],),
                device_id_type=pl.DeviceIdType.MESH,
            )

        rdmas = {}

        for q in range(N_SUB):
            for d in ("a", "b"):
                rows = dirs[d]["base"] + q * sub
                cp = pltpu.make_async_copy(
                    x_ref.at[pl.ds(rows, sub), :], stage, stage_sem)
                cp.start()
                cp.wait()
                dirs[d]["buf"][0, pl.ds(q * sub, sub), :] = (
                    stage[...].astype(jnp.float8_e4m3fn))
                r = make_rdma(d, 0, q)
                r.start()
                rdmas[(d, 0, q)] = r

        w16[...] = w_ref[...].astype(jnp.bfloat16)

        scale = sx_ref[0, 0] * sw_ref[0, 0]
        out_copies = []

        def compute_block(d, s, row0, nrows):
            dd = dirs[d]
            blk = len(out_copies)
            slot = blk % 2
            if blk >= 2:
                out_copies[blk - 2].wait()
            acc = jnp.dot(
                dd["buf"][s, pl.ds(row0, nrows), :].astype(jnp.bfloat16),
                w16[...], preferred_element_type=jnp.float32)
            outblk[slot, pl.ds(0, nrows), :] = jnp.maximum(acc * scale, 0.0)
            base = dd["origin"](s) * m_per + dd["base"]
            cp = pltpu.make_async_copy(
                outblk.at[slot, pl.ds(0, nrows), :],
                out_ref.at[pl.ds(base + row0, nrows), :],
                out_sems.at[slot],
            )
            cp.start()
            out_copies.append(cp)

        def compute_slot(s):
            compute_block("a", s, 0, half)
            compute_block("b", s, 0, half)

        compute_slot(0)

        for h in range(1, N_HOP):
            for q in range(N_SUB):
                for d in ("a", "b"):
                    rdmas[(d, h - 1, q)].wait()
                    r = make_rdma(d, h, q)
                    r.start()
                    rdmas[(d, h, q)] = r
            compute_slot(h)

        for q in range(N_SUB):
            for d in ("a", "b"):
                rdmas[(d, N_HOP - 1, q)].wait()
            for d in ("a", "b"):
                compute_block(d, N_DEV - 1, q * sub, sub)

        out_copies[-2].wait()
        out_copies[-1].wait()

    return pl.pallas_call(
        body,
        out_shape=jax.ShapeDtypeStruct((N_DEV * m_per, n_per), jnp.float32),
        in_specs=[
            pl.BlockSpec(memory_space=pl.ANY),
            pl.BlockSpec(memory_space=pltpu.VMEM),
            pl.BlockSpec(memory_space=pltpu.SMEM),
            pl.BlockSpec(memory_space=pltpu.SMEM),
        ],
        out_specs=pl.BlockSpec(memory_space=pl.ANY),
        scratch_shapes=[
            pltpu.VMEM((N_DEV, half, k), jnp.float8_e4m3fn),
            pltpu.VMEM((N_DEV, half, k), jnp.float8_e4m3fn),
            pltpu.VMEM((sub, k), jnp.float32),
            pltpu.VMEM((k, n_per), jnp.bfloat16),
            pltpu.VMEM((2, half, n_per), jnp.float32),
            pltpu.SemaphoreType.DMA,
            pltpu.SemaphoreType.DMA((2,)),
        ] + [
            pltpu.SemaphoreType.DMA((N_HOP,))
            for _ in range(4 * N_SUB)
        ],
        compiler_params=pltpu.CompilerParams(
            collective_id=0,
            vmem_limit_bytes=100 * 1024 * 1024,
        ),
    )(x, w_mat, scale_x.reshape(1, 1), scale_w.reshape(1, 1))


# device time: 55056 ns/iter; 1.7458x vs baseline; 1.6489x over previous
import jax
import jax.numpy as jnp
from jax import lax
from jax.experimental import pallas as pl
from jax.experimental.pallas import tpu as pltpu

N_DEV = 4
N_HOP = N_DEV - 1
N_SUB = 4


def kernel(x, w_mat, scale_x, scale_w):
    m_per, k = x.shape
    _, n_per = w_mat.shape
    half = m_per // 2
    sub = half // N_SUB

    def body(x_ref, w_ref, sx_ref, sw_ref, out_ref,
             buf_a, buf_b, stage, w16, outblk,
             stage_sem, out_sems, *ring_sems):
        my = lax.axis_index("i")
        left = lax.rem(my + N_DEV - 1, N_DEV)
        right = lax.rem(my + 1, N_DEV)

        dirs = {
            "a": dict(buf=buf_a, peer=right, base=0,
                      origin=lambda s: lax.rem(my - s + N_DEV, N_DEV)),
            "b": dict(buf=buf_b, peer=left, base=half,
                      origin=lambda s: lax.rem(my + s, N_DEV)),
        }
        for q in range(N_SUB):
            for di, d in enumerate(("a", "b")):
                dirs[d].setdefault("send", {})[q] = ring_sems[4 * q + 2 * di]
                dirs[d].setdefault("recv", {})[q] = ring_sems[4 * q + 2 * di + 1]

        barrier = pltpu.get_barrier_semaphore()
        for nbr in (left, right):
            pl.semaphore_signal(
                barrier, inc=1,
                device_id=(nbr,), device_id_type=pl.DeviceIdType.MESH,
            )
        pl.semaphore_wait(barrier, 2)

        def make_rdma(d, h, q):
            dd = dirs[d]
            return pltpu.make_async_remote_copy(
                src_ref=dd["buf"].at[h, pl.ds(q * sub, sub), :],
                dst_ref=dd["buf"].at[h + 1, pl.ds(q * sub, sub), :],
                send_sem=dd["send"][q].at[h],
                recv_sem=dd["recv"][q].at[h],
                device_id=(dd["peer"],),
                device_id_type=pl.DeviceIdType.MESH,
            )

        rdmas = {}

        for q in range(N_SUB):
            for d in ("a", "b"):
                rows = dirs[d]["base"] + q * sub
                cp = pltpu.make_async_copy(
                    x_ref.at[pl.ds(rows, sub), :], stage, stage_sem)
                cp.start()
                cp.wait()
                dirs[d]["buf"][0, pl.ds(q * sub, sub), :] = (
                    stage[...].astype(jnp.float8_e4m3fn))
                r = make_rdma(d, 0, q)
                r.start()
                rdmas[(d, 0, q)] = r

        w16[...] = w_ref[...].astype(jnp.bfloat16)

        scale = sx_ref[0, 0] * sw_ref[0, 0]
        out_copies = []

        def compute_block(d, s, row0, nrows):
            dd = dirs[d]
            blk = len(out_copies)
            slot = blk % 2
            if blk >= 2:
                out_copies[blk - 2].wait()
            acc = jnp.dot(
                dd["buf"][s, pl.ds(row0, nrows), :].astype(jnp.bfloat16),
                w16[...], preferred_element_type=jnp.float32)
            outblk[slot, pl.ds(0, nrows), :] = jnp.maximum(acc * scale, 0.0)
            base = dd["origin"](s) * m_per + dd["base"]
            cp = pltpu.make_async_copy(
                outblk.at[slot, pl.ds(0, nrows), :],
                out_ref.at[pl.ds(base + row0, nrows), :],
                out_sems.at[slot],
            )
            cp.start()
            out_copies.append(cp)

        def compute_slot(s):
            compute_block("a", s, 0, half)
            compute_block("b", s, 0, half)

        compute_slot(0)

        for h in range(1, N_HOP):
            for q in range(N_SUB):
                for d in ("a", "b"):
                    rdmas[(d, h - 1, q)].wait()
                    r = make_rdma(d, h, q)
                    r.start()
                    rdmas[(d, h, q)] = r
            compute_slot(h)

        for q in range(N_SUB):
            for d in ("a", "b"):
                rdmas[(d, N_HOP - 1, q)].wait()
            for d in ("a", "b"):
                compute_block(d, N_DEV - 1, q * sub, sub)

        out_copies[-2].wait()
        out_copies[-1].wait()

    return pl.pallas_call(
        body,
        out_shape=jax.ShapeDtypeStruct((N_DEV * m_per, n_per), jnp.float32),
        in_specs=[
            pl.BlockSpec(memory_space=pl.ANY),
            pl.BlockSpec(memory_space=pltpu.VMEM),
            pl.BlockSpec(memory_space=pltpu.SMEM),
            pl.BlockSpec(memory_space=pltpu.SMEM),
        ],
        out_specs=pl.BlockSpec(memory_space=pl.ANY),
        scratch_shapes=[
            pltpu.VMEM((N_DEV, half, k), jnp.float8_e4m3fn),
            pltpu.VMEM((N_DEV, half, k), jnp.float8_e4m3fn),
            pltpu.VMEM((sub, k), jnp.float32),
            pltpu.VMEM((k, n_per), jnp.bfloat16),
            pltpu.VMEM((2, half, n_per), jnp.float32),
            pltpu.SemaphoreType.DMA,
            pltpu.SemaphoreType.DMA((2,)),
        ] + [
            pltpu.SemaphoreType.DMA((N_HOP,))
            for _ in range(4 * N_SUB)
        ],
        compiler_params=pltpu.CompilerParams(
            collective_id=0,
            vmem_limit_bytes=100 * 1024 * 1024,
        ),
    )(x, w_mat, scale_x.reshape(1, 1), scale_w.reshape(1, 1))
